# baseline (speedup 1.0000x reference)
import functools

import jax
import jax.numpy as jnp
from jax import lax
from jax.experimental import pallas as pl
from jax.experimental.pallas import tpu as pltpu
from jax.experimental.pallas import tpu_sc as plsc

NC = 2
NS = 16


def _dense1_body(x_ref, ws_ref, wn_ref, b_ref, s_ref, p_ref):
    xb = x_ref[...]
    s = jnp.dot(xb, ws_ref[...], preferred_element_type=jnp.float32) + b_ref[...]
    p = jnp.dot(xb, wn_ref[...], preferred_element_type=jnp.float32)
    h = s.shape[1] // 2
    s_ref[0] = s[:, :h]
    s_ref[1] = s[:, h:]
    p_ref[0] = p[:, :h]
    p_ref[1] = p[:, h:]


def _dense1(x, w_self, w_neigh, b, bn):
    n, d_in = x.shape
    d_out = w_self.shape[1]
    h = d_out // 2
    grid = (n // bn,)
    out_shape = jax.ShapeDtypeStruct((2, n, h), jnp.float32)
    s, p = pl.pallas_call(
        _dense1_body,
        grid=grid,
        in_specs=[
            pl.BlockSpec((bn, d_in), lambda i: (i, 0)),
            pl.BlockSpec((d_in, d_out), lambda i: (0, 0)),
            pl.BlockSpec((d_in, d_out), lambda i: (0, 0)),
            pl.BlockSpec((1, d_out), lambda i: (0, 0)),
        ],
        out_specs=[
            pl.BlockSpec((2, bn, h), lambda i: (0, i, 0)),
            pl.BlockSpec((2, bn, h), lambda i: (0, i, 0)),
        ],
        out_shape=[out_shape, out_shape],
    )(x, w_self, w_neigh, b.reshape(1, d_out))
    return s.reshape(2 * n, h), p.reshape(2 * n, h)


def _dense2_body(h_ref, ws_ref, wn_ref, b_ref, s_ref, p_ref):
    h0 = jnp.maximum(h_ref[0], 0.0)
    h1 = jnp.maximum(h_ref[1], 0.0)
    s = (jnp.dot(h0, ws_ref[0], preferred_element_type=jnp.float32)
         + jnp.dot(h1, ws_ref[1], preferred_element_type=jnp.float32)
         + b_ref[...])
    p = (jnp.dot(h0, wn_ref[0], preferred_element_type=jnp.float32)
         + jnp.dot(h1, wn_ref[1], preferred_element_type=jnp.float32))
    s_ref[0] = s
    s_ref[1] = jnp.zeros_like(s)
    p_ref[...] = p


def _dense2(hpre, w_self, w_neigh, b, bn):
    _, n, dh = hpre.shape
    d_out = w_self.shape[2]
    grid = (n // bn,)
    s, p = pl.pallas_call(
        _dense2_body,
        grid=grid,
        in_specs=[
            pl.BlockSpec((2, bn, dh), lambda i: (0, i, 0)),
            pl.BlockSpec((2, dh, d_out), lambda i: (0, 0, 0)),
            pl.BlockSpec((2, dh, d_out), lambda i: (0, 0, 0)),
            pl.BlockSpec((1, d_out), lambda i: (0, 0)),
        ],
        out_specs=[
            pl.BlockSpec((2, bn, d_out), lambda i: (0, i, 0)),
            pl.BlockSpec((bn, d_out), lambda i: (i, 0)),
        ],
        out_shape=[
            jax.ShapeDtypeStruct((2, n, d_out), jnp.float32),
            jax.ShapeDtypeStruct((n, d_out), jnp.float32),
        ],
    )(hpre, w_self, w_neigh, b.reshape(1, d_out))
    return s.reshape(2 * n, d_out), p


def _combine_body(a_ref, b_ref, o_ref):
    o_ref[...] = a_ref[...] + b_ref[...]


def _combine(o2, n, d_out, bn):
    return pl.pallas_call(
        _combine_body,
        grid=(n // bn,),
        in_specs=[
            pl.BlockSpec((bn, d_out), lambda i: (i, 0)),
            pl.BlockSpec((bn, d_out), lambda i: (i + n // bn, 0)),
        ],
        out_specs=pl.BlockSpec((bn, d_out), lambda i: (i, 0)),
        out_shape=jax.ShapeDtypeStruct((n, d_out), jnp.float32),
    )(o2, o2)


def _make_sc_agg(n, e_per_core, dh, chunk, snd_stride, rcv_stride, nph=1):
    ept = e_per_core // NS
    ept_ph = ept // nph
    nch_ph = ept_ph // chunk
    assert ept_ph % chunk == 0 and ept_ph % 8 == 0 and chunk % 8 == 0
    assert chunk <= 128
    npad = 16
    rows_a = ((n // NS + 7) // 8) * 8
    rows_last = n - rows_a * (NS - 1)
    mesh = plsc.VectorSubcoreMesh(core_axis_name="c", subcore_axis_name="s")

    @functools.partial(
        pl.kernel,
        out_type=jax.ShapeDtypeStruct((2 * n, dh), jnp.float32),
        mesh=mesh,
        scratch_types=[
            pltpu.VMEM((ept_ph,), jnp.int32),
            pltpu.VMEM((ept_ph,), jnp.int32),
            pltpu.VMEM((chunk, dh), jnp.float32),
            pltpu.VMEM_SHARED((n + npad, dh), jnp.float32),
        ],
    )
    def sc_agg(p_hbm, s_hbm, snd_hbm, rcv_hbm, out_hbm,
               snd_v, rcv_v, buf0, acc):
        c = lax.axis_index("c")
        t = lax.axis_index("s")
        base = t * rows_a

        @pl.when(t < NS - 1)
        def _():
            pltpu.sync_copy(s_hbm.at[pl.ds(c * n + base, rows_a)],
                            acc.at[pl.ds(base, rows_a)])

        @pl.when(t == NS - 1)
        def _():
            pltpu.sync_copy(s_hbm.at[pl.ds(c * n + base, rows_last)],
                            acc.at[pl.ds(base, rows_last)])

        plsc.subcore_barrier()

        for ph in range(nph):
            pltpu.sync_copy(
                snd_hbm.at[pl.ds(c * snd_stride + t * ept + ph * ept_ph,
                                 ept_ph)], snd_v)
            pltpu.sync_copy(
                rcv_hbm.at[pl.ds(c * rcv_stride + t * ept + ph * ept_ph,
                                 ept_ph)], rcv_v)

            @pl.loop(0, nch_ph)
            def _(j):
                pltpu.sync_copy(p_hbm.at[snd_v.at[pl.ds(j * chunk, chunk)]],
                                buf0)
                pltpu.sync_copy(buf0,
                                acc.at[rcv_v.at[pl.ds(j * chunk, chunk)]],
                                add=True)

        plsc.subcore_barrier()

        @pl.when(t < NS - 1)
        def _():
            pltpu.sync_copy(acc.at[pl.ds(base, rows_a)],
                            out_hbm.at[pl.ds(c * n + base, rows_a)])

        @pl.when(t == NS - 1)
        def _():
            pltpu.sync_copy(acc.at[pl.ds(base, rows_last)],
                            out_hbm.at[pl.ds(c * n + base, rows_last)])

    return sc_agg


def kernel(x, senders, receivers, W1_self, W1_neigh, b1, W2_self, W2_neigh, b2):
    n, d_in = x.shape
    d_hid = W1_self.shape[1]
    d_out = W2_self.shape[1]
    e = senders.shape[0]
    bn = 1000
    chunk1, chunk2 = 40, 40

    epad = e + ((-e) % (2 * NS * max(chunk1, 2 * chunk2)))
    pad = epad - e
    snd_p = jnp.concatenate(
        [senders.astype(jnp.int32), jnp.zeros((pad,), jnp.int32)])
    rcv_p = jnp.concatenate(
        [receivers.astype(jnp.int32),
         n + (jnp.arange(pad, dtype=jnp.int32) % 16)])
    snd2 = jnp.concatenate([snd_p, snd_p + n])

    s1, p1 = _dense1(x, W1_self, W1_neigh, b1, bn)
    hpre = _make_sc_agg(n, epad, d_hid // 2, chunk1, snd_stride=epad,
                        rcv_stride=0)(p1, s1, snd2, rcv_p)
    s2, p2 = _dense2(hpre.reshape(2, n, d_hid // 2),
                     W2_self.reshape(2, d_hid // 2, d_out),
                     W2_neigh.reshape(2, d_hid // 2, d_out), b2, bn)
    e2 = epad // 2
    o2 = _make_sc_agg(n, e2, d_out, chunk2, snd_stride=e2, rcv_stride=e2)(
        p2, s2, snd_p, rcv_p)
    return _combine(o2, n, d_out, bn)


# speedup vs baseline: 1.1528x; 1.1528x over previous
import functools

import jax
import jax.numpy as jnp
from jax import lax
from jax.experimental import pallas as pl
from jax.experimental.pallas import tpu as pltpu
from jax.experimental.pallas import tpu_sc as plsc

NC = 2
NS = 16


def _dense1_body(x_ref, ws_ref, wn_ref, b_ref, s_ref, p_ref):
    xb = x_ref[...]
    s = jnp.dot(xb, ws_ref[...], preferred_element_type=jnp.float32) + b_ref[...]
    p = jnp.dot(xb, wn_ref[...], preferred_element_type=jnp.float32)
    h = s.shape[1] // 2
    s_ref[0] = s[:, :h]
    s_ref[1] = s[:, h:]
    p_ref[0] = p[:, :h]
    p_ref[1] = p[:, h:]


def _dense1(x, w_self, w_neigh, b, bn):
    n, d_in = x.shape
    d_out = w_self.shape[1]
    h = d_out // 2
    grid = (n // bn,)
    out_shape = jax.ShapeDtypeStruct((2, n, h), jnp.float32)
    s, p = pl.pallas_call(
        _dense1_body,
        grid=grid,
        in_specs=[
            pl.BlockSpec((bn, d_in), lambda i: (i, 0)),
            pl.BlockSpec((d_in, d_out), lambda i: (0, 0)),
            pl.BlockSpec((d_in, d_out), lambda i: (0, 0)),
            pl.BlockSpec((1, d_out), lambda i: (0, 0)),
        ],
        out_specs=[
            pl.BlockSpec((2, bn, h), lambda i: (0, i, 0)),
            pl.BlockSpec((2, bn, h), lambda i: (0, i, 0)),
        ],
        out_shape=[out_shape, out_shape],
    )(x, w_self, w_neigh, b.reshape(1, d_out))
    return s.reshape(2 * n, h), p.reshape(2 * n, h)


def _dense2_body(h_ref, ws_ref, wn_ref, b_ref, s_ref, p_ref):
    h0 = jnp.maximum(h_ref[0], 0.0)
    h1 = jnp.maximum(h_ref[1], 0.0)
    s = (jnp.dot(h0, ws_ref[0], preferred_element_type=jnp.float32)
         + jnp.dot(h1, ws_ref[1], preferred_element_type=jnp.float32)
         + b_ref[...])
    p = (jnp.dot(h0, wn_ref[0], preferred_element_type=jnp.float32)
         + jnp.dot(h1, wn_ref[1], preferred_element_type=jnp.float32))
    s_ref[0] = s
    s_ref[1] = jnp.zeros_like(s)
    p_ref[...] = p


def _dense2(hpre, w_self, w_neigh, b, bn):
    _, n, dh = hpre.shape
    d_out = w_self.shape[2]
    grid = (n // bn,)
    s, p = pl.pallas_call(
        _dense2_body,
        grid=grid,
        in_specs=[
            pl.BlockSpec((2, bn, dh), lambda i: (0, i, 0)),
            pl.BlockSpec((2, dh, d_out), lambda i: (0, 0, 0)),
            pl.BlockSpec((2, dh, d_out), lambda i: (0, 0, 0)),
            pl.BlockSpec((1, d_out), lambda i: (0, 0)),
        ],
        out_specs=[
            pl.BlockSpec((2, bn, d_out), lambda i: (0, i, 0)),
            pl.BlockSpec((bn, d_out), lambda i: (i, 0)),
        ],
        out_shape=[
            jax.ShapeDtypeStruct((2, n, d_out), jnp.float32),
            jax.ShapeDtypeStruct((n, d_out), jnp.float32),
        ],
    )(hpre, w_self, w_neigh, b.reshape(1, d_out))
    return s.reshape(2 * n, d_out), p


def _combine_body(a_ref, b_ref, o_ref):
    o_ref[...] = a_ref[...] + b_ref[...]


def _combine(o2, n, d_out, bn):
    return pl.pallas_call(
        _combine_body,
        grid=(n // bn,),
        in_specs=[
            pl.BlockSpec((bn, d_out), lambda i: (i, 0)),
            pl.BlockSpec((bn, d_out), lambda i: (i + n // bn, 0)),
        ],
        out_specs=pl.BlockSpec((bn, d_out), lambda i: (i, 0)),
        out_shape=jax.ShapeDtypeStruct((n, d_out), jnp.float32),
    )(o2, o2)


def _make_sc_agg(n, e_per_core, dh, chunk, snd_stride, rcv_stride, nph=1):
    ept = e_per_core // NS
    ept_ph = ept // nph
    nch_ph = ept_ph // chunk
    assert ept_ph % chunk == 0 and ept_ph % 8 == 0 and chunk % 8 == 0
    assert chunk <= 128
    npad = 16
    rows_a = ((n // NS + 7) // 8) * 8
    rows_last = n - rows_a * (NS - 1)
    mesh = plsc.VectorSubcoreMesh(core_axis_name="c", subcore_axis_name="s")

    @functools.partial(
        pl.kernel,
        out_type=jax.ShapeDtypeStruct((2 * n, dh), jnp.float32),
        mesh=mesh,
        scratch_types=[
            pltpu.VMEM((ept_ph,), jnp.int32),
            pltpu.VMEM((ept_ph,), jnp.int32),
            pltpu.VMEM((chunk, dh), jnp.float32),
            pltpu.VMEM_SHARED((n + npad, dh), jnp.float32),
        ],
    )
    def sc_agg(p_hbm, s_hbm, snd_hbm, rcv_hbm, out_hbm,
               snd_v, rcv_v, buf0, acc):
        c = lax.axis_index("c")
        t = lax.axis_index("s")
        base = t * rows_a

        @pl.when(t < NS - 1)
        def _():
            pltpu.sync_copy(s_hbm.at[pl.ds(c * n + base, rows_a)],
                            acc.at[pl.ds(base, rows_a)])

        @pl.when(t == NS - 1)
        def _():
            pltpu.sync_copy(s_hbm.at[pl.ds(c * n + base, rows_last)],
                            acc.at[pl.ds(base, rows_last)])

        plsc.subcore_barrier()

        for ph in range(nph):
            pltpu.sync_copy(
                snd_hbm.at[pl.ds(c * snd_stride + t * ept + ph * ept_ph,
                                 ept_ph)], snd_v)
            pltpu.sync_copy(
                rcv_hbm.at[pl.ds(c * rcv_stride + t * ept + ph * ept_ph,
                                 ept_ph)], rcv_v)

            @pl.loop(0, nch_ph)
            def _(j):
                pltpu.sync_copy(p_hbm.at[snd_v.at[pl.ds(j * chunk, chunk)]],
                                buf0)
                pltpu.sync_copy(buf0,
                                acc.at[rcv_v.at[pl.ds(j * chunk, chunk)]],
                                add=True)

        plsc.subcore_barrier()

        @pl.when(t < NS - 1)
        def _():
            pltpu.sync_copy(acc.at[pl.ds(base, rows_a)],
                            out_hbm.at[pl.ds(c * n + base, rows_a)])

        @pl.when(t == NS - 1)
        def _():
            pltpu.sync_copy(acc.at[pl.ds(base, rows_last)],
                            out_hbm.at[pl.ds(c * n + base, rows_last)])

    return sc_agg


def kernel(x, senders, receivers, W1_self, W1_neigh, b1, W2_self, W2_neigh, b2):
    n, d_in = x.shape
    d_hid = W1_self.shape[1]
    d_out = W2_self.shape[1]
    e = senders.shape[0]
    bn = 1000
    chunk1, chunk2 = 80, 40

    epad = e + ((-e) % (2 * NS * max(chunk1, 2 * chunk2)))
    pad = epad - e
    snd_p = jnp.concatenate(
        [senders.astype(jnp.int32), jnp.zeros((pad,), jnp.int32)])
    rcv_p = jnp.concatenate(
        [receivers.astype(jnp.int32),
         n + (jnp.arange(pad, dtype=jnp.int32) % 16)])
    snd2 = jnp.concatenate([snd_p, snd_p + n])

    s1, p1 = _dense1(x, W1_self, W1_neigh, b1, bn)
    hpre = _make_sc_agg(n, epad, d_hid // 2, chunk1, snd_stride=epad,
                        rcv_stride=0)(p1, s1, snd2, rcv_p)
    s2, p2 = _dense2(hpre.reshape(2, n, d_hid // 2),
                     W2_self.reshape(2, d_hid // 2, d_out),
                     W2_neigh.reshape(2, d_hid // 2, d_out), b2, bn)
    e2 = epad // 2
    o2 = _make_sc_agg(n, e2, d_out, chunk2, snd_stride=e2, rcv_stride=e2)(
        p2, s2, snd_p, rcv_p)
    return _combine(o2, n, d_out, bn)


# speedup vs baseline: 1.1662x; 1.0116x over previous
import functools

import jax
import jax.numpy as jnp
from jax import lax
from jax.experimental import pallas as pl
from jax.experimental.pallas import tpu as pltpu
from jax.experimental.pallas import tpu_sc as plsc

NC = 2
NS = 16


def _dense1_body(x_ref, ws_ref, wn_ref, b_ref, s_ref, p_ref):
    xb = x_ref[...]
    s = jnp.dot(xb, ws_ref[...], preferred_element_type=jnp.float32) + b_ref[...]
    p = jnp.dot(xb, wn_ref[...], preferred_element_type=jnp.float32)
    h = s.shape[1] // 2
    s_ref[0] = s[:, :h]
    s_ref[1] = s[:, h:]
    p_ref[0] = p[:, :h]
    p_ref[1] = p[:, h:]


def _dense1(x, w_self, w_neigh, b, bn):
    n, d_in = x.shape
    d_out = w_self.shape[1]
    h = d_out // 2
    grid = (n // bn,)
    out_shape = jax.ShapeDtypeStruct((2, n, h), jnp.float32)
    s, p = pl.pallas_call(
        _dense1_body,
        grid=grid,
        in_specs=[
            pl.BlockSpec((bn, d_in), lambda i: (i, 0)),
            pl.BlockSpec((d_in, d_out), lambda i: (0, 0)),
            pl.BlockSpec((d_in, d_out), lambda i: (0, 0)),
            pl.BlockSpec((1, d_out), lambda i: (0, 0)),
        ],
        out_specs=[
            pl.BlockSpec((2, bn, h), lambda i: (0, i, 0)),
            pl.BlockSpec((2, bn, h), lambda i: (0, i, 0)),
        ],
        out_shape=[out_shape, out_shape],
    )(x, w_self, w_neigh, b.reshape(1, d_out))
    return s.reshape(2 * n, h), p.reshape(2 * n, h)


def _dense2_body(h_ref, ws_ref, wn_ref, b_ref, s_ref, p_ref):
    h0 = jnp.maximum(h_ref[0], 0.0)
    h1 = jnp.maximum(h_ref[1], 0.0)
    s = (jnp.dot(h0, ws_ref[0], preferred_element_type=jnp.float32)
         + jnp.dot(h1, ws_ref[1], preferred_element_type=jnp.float32)
         + b_ref[...])
    p = (jnp.dot(h0, wn_ref[0], preferred_element_type=jnp.float32)
         + jnp.dot(h1, wn_ref[1], preferred_element_type=jnp.float32))
    s_ref[0] = s
    s_ref[1] = jnp.zeros_like(s)
    p_ref[...] = p


def _dense2(hpre, w_self, w_neigh, b, bn):
    _, n, dh = hpre.shape
    d_out = w_self.shape[2]
    grid = (n // bn,)
    s, p = pl.pallas_call(
        _dense2_body,
        grid=grid,
        in_specs=[
            pl.BlockSpec((2, bn, dh), lambda i: (0, i, 0)),
            pl.BlockSpec((2, dh, d_out), lambda i: (0, 0, 0)),
            pl.BlockSpec((2, dh, d_out), lambda i: (0, 0, 0)),
            pl.BlockSpec((1, d_out), lambda i: (0, 0)),
        ],
        out_specs=[
            pl.BlockSpec((2, bn, d_out), lambda i: (0, i, 0)),
            pl.BlockSpec((bn, d_out), lambda i: (i, 0)),
        ],
        out_shape=[
            jax.ShapeDtypeStruct((2, n, d_out), jnp.float32),
            jax.ShapeDtypeStruct((n, d_out), jnp.float32),
        ],
    )(hpre, w_self, w_neigh, b.reshape(1, d_out))
    return s.reshape(2 * n, d_out), p


def _combine_body(a_ref, b_ref, o_ref):
    o_ref[...] = a_ref[...] + b_ref[...]


def _combine(o2, n, d_out, bn):
    return pl.pallas_call(
        _combine_body,
        grid=(n // bn,),
        in_specs=[
            pl.BlockSpec((bn, d_out), lambda i: (i, 0)),
            pl.BlockSpec((bn, d_out), lambda i: (i + n // bn, 0)),
        ],
        out_specs=pl.BlockSpec((bn, d_out), lambda i: (i, 0)),
        out_shape=jax.ShapeDtypeStruct((n, d_out), jnp.float32),
    )(o2, o2)


def _make_sc_agg(n, e_per_core, dh, chunk, snd_stride, rcv_stride, nph=1):
    ept = e_per_core // NS
    ept_ph = ept // nph
    nch_ph = ept_ph // chunk
    assert ept_ph % chunk == 0 and ept_ph % 8 == 0 and chunk % 8 == 0
    assert chunk <= 128
    npad = 16
    rows_a = ((n // NS + 7) // 8) * 8
    rows_last = n - rows_a * (NS - 1)
    mesh = plsc.VectorSubcoreMesh(core_axis_name="c", subcore_axis_name="s")

    @functools.partial(
        pl.kernel,
        out_type=jax.ShapeDtypeStruct((2 * n, dh), jnp.float32),
        mesh=mesh,
        scratch_types=[
            pltpu.VMEM((ept_ph,), jnp.int32),
            pltpu.VMEM((ept_ph,), jnp.int32),
            pltpu.VMEM((chunk, dh), jnp.float32),
            pltpu.VMEM_SHARED((n + npad, dh), jnp.float32),
        ],
    )
    def sc_agg(p_hbm, s_hbm, snd_hbm, rcv_hbm, out_hbm,
               snd_v, rcv_v, buf0, acc):
        c = lax.axis_index("c")
        t = lax.axis_index("s")
        base = t * rows_a

        @pl.when(t < NS - 1)
        def _():
            pltpu.sync_copy(s_hbm.at[pl.ds(c * n + base, rows_a)],
                            acc.at[pl.ds(base, rows_a)])

        @pl.when(t == NS - 1)
        def _():
            pltpu.sync_copy(s_hbm.at[pl.ds(c * n + base, rows_last)],
                            acc.at[pl.ds(base, rows_last)])

        plsc.subcore_barrier()

        for ph in range(nph):
            pltpu.sync_copy(
                snd_hbm.at[pl.ds(c * snd_stride + t * ept + ph * ept_ph,
                                 ept_ph)], snd_v)
            pltpu.sync_copy(
                rcv_hbm.at[pl.ds(c * rcv_stride + t * ept + ph * ept_ph,
                                 ept_ph)], rcv_v)

            @pl.loop(0, nch_ph)
            def _(j):
                pltpu.sync_copy(p_hbm.at[snd_v.at[pl.ds(j * chunk, chunk)]],
                                buf0)
                pltpu.sync_copy(buf0,
                                acc.at[rcv_v.at[pl.ds(j * chunk, chunk)]],
                                add=True)

        plsc.subcore_barrier()

        @pl.when(t < NS - 1)
        def _():
            pltpu.sync_copy(acc.at[pl.ds(base, rows_a)],
                            out_hbm.at[pl.ds(c * n + base, rows_a)])

        @pl.when(t == NS - 1)
        def _():
            pltpu.sync_copy(acc.at[pl.ds(base, rows_last)],
                            out_hbm.at[pl.ds(c * n + base, rows_last)])

    return sc_agg


def kernel(x, senders, receivers, W1_self, W1_neigh, b1, W2_self, W2_neigh, b2):
    n, d_in = x.shape
    d_hid = W1_self.shape[1]
    d_out = W2_self.shape[1]
    e = senders.shape[0]
    bn = 2000
    chunk1, chunk2 = 80, 40

    epad = e + ((-e) % (2 * NS * max(chunk1, 2 * chunk2)))
    pad = epad - e
    snd_p = jnp.concatenate(
        [senders.astype(jnp.int32), jnp.zeros((pad,), jnp.int32)])
    rcv_p = jnp.concatenate(
        [receivers.astype(jnp.int32),
         n + (jnp.arange(pad, dtype=jnp.int32) % 16)])
    snd2 = jnp.concatenate([snd_p, snd_p + n])

    s1, p1 = _dense1(x, W1_self, W1_neigh, b1, bn)
    hpre = _make_sc_agg(n, epad, d_hid // 2, chunk1, snd_stride=epad,
                        rcv_stride=0)(p1, s1, snd2, rcv_p)
    s2, p2 = _dense2(hpre.reshape(2, n, d_hid // 2),
                     W2_self.reshape(2, d_hid // 2, d_out),
                     W2_neigh.reshape(2, d_hid // 2, d_out), b2, bn)
    e2 = epad // 2
    o2 = _make_sc_agg(n, e2, d_out, chunk2, snd_stride=e2, rcv_stride=e2)(
        p2, s2, snd_p, rcv_p)
    return _combine(o2, n, d_out, bn)
